# baseline (speedup 1.0000x reference)
import functools

import jax
import jax.numpy as jnp
from jax.experimental import pallas as pl
from jax.experimental.pallas import tpu as pltpu

_EPS = 1e-20
_INT_MAX = 2**31 - 1


def _threefry2x32_bits(i_u32):
    ks0 = jnp.uint32(0)
    ks1 = jnp.uint32(1)
    ks2 = jnp.uint32(0x1BD11BDA) ^ ks0 ^ ks1

    x0 = jnp.zeros_like(i_u32) + ks0
    x1 = i_u32 + ks1

    rot0 = (13, 15, 26, 6)
    rot1 = (17, 29, 16, 24)

    def rounds(x0, x1, rots):
        for d in rots:
            x0 = x0 + x1
            x1 = (x1 << jnp.uint32(d)) | (x1 >> jnp.uint32(32 - d))
            x1 = x0 ^ x1
        return x0, x1

    x0, x1 = rounds(x0, x1, rot0)
    x0 = x0 + ks1
    x1 = x1 + ks2 + jnp.uint32(1)
    x0, x1 = rounds(x0, x1, rot1)
    x0 = x0 + ks2
    x1 = x1 + ks0 + jnp.uint32(2)
    x0, x1 = rounds(x0, x1, rot0)
    x0 = x0 + ks0
    x1 = x1 + ks1 + jnp.uint32(3)
    x0, x1 = rounds(x0, x1, rot1)
    x0 = x0 + ks1
    x1 = x1 + ks2 + jnp.uint32(4)
    x0, x1 = rounds(x0, x1, rot0)
    x0 = x0 + ks2
    x1 = x1 + ks0 + jnp.uint32(5)
    return x0 ^ x1


def _argmax_body(n_cols, blk, x_ref, out_ref, bestv_ref, besti_ref):
    j = pl.program_id(0)
    nb = pl.num_programs(0)
    rows = x_ref.shape[0]

    @pl.when(j == 0)
    def _init():
        bestv_ref[...] = jnp.full_like(bestv_ref, -jnp.inf)
        besti_ref[...] = jnp.zeros_like(besti_ref)

    c = jax.lax.broadcasted_iota(jnp.int32, (rows, blk), 1) + j * blk
    r = jax.lax.broadcasted_iota(jnp.int32, (rows, blk), 0)
    flat = (r * n_cols + c).astype(jnp.uint32)

    bits = _threefry2x32_bits(flat)
    fbits = (bits >> jnp.uint32(9)) | jnp.uint32(0x3F800000)
    u = jax.lax.bitcast_convert_type(fbits, jnp.float32) - jnp.float32(1.0)
    g = -jnp.log(-jnp.log(u + _EPS) + _EPS)

    y = x_ref[...] + g
    y = jnp.where(c < n_cols, y, -jnp.inf)

    m = jnp.max(y, axis=1, keepdims=True)
    idx = jnp.min(jnp.where(y == m, c, _INT_MAX), axis=1, keepdims=True)

    upd = m > bestv_ref[...]
    bestv_ref[...] = jnp.where(upd, m, bestv_ref[...])
    besti_ref[...] = jnp.where(upd, idx, besti_ref[...])

    @pl.when(j == nb - 1)
    def _fin():
        out_ref[...] = besti_ref[...]


def _onehot_body(blk, idx_ref, out_ref):
    j = pl.program_id(0)
    rows = out_ref.shape[0]
    c = jax.lax.broadcasted_iota(jnp.int32, (rows, blk), 1) + j * blk
    out_ref[...] = jnp.where(c == idx_ref[...], jnp.float32(1.0),
                             jnp.float32(0.0))


def kernel(input):
    rows, n_cols = input.shape

    blk_a = 8192
    nb_a = pl.cdiv(n_cols, blk_a)
    idx = pl.pallas_call(
        functools.partial(_argmax_body, n_cols, blk_a),
        grid=(nb_a,),
        in_specs=[pl.BlockSpec((rows, blk_a), lambda j: (0, j))],
        out_specs=pl.BlockSpec((rows, 1), lambda j: (0, 0)),
        out_shape=jax.ShapeDtypeStruct((rows, 1), jnp.int32),
        scratch_shapes=[
            pltpu.VMEM((rows, 1), jnp.float32),
            pltpu.VMEM((rows, 1), jnp.int32),
        ],
    )(input)

    blk_b = 8192
    nb_b = pl.cdiv(n_cols, blk_b)
    out = pl.pallas_call(
        functools.partial(_onehot_body, blk_b),
        grid=(nb_b,),
        in_specs=[pl.BlockSpec((rows, 1), lambda j: (0, 0))],
        out_specs=pl.BlockSpec((rows, blk_b), lambda j: (0, j)),
        out_shape=jax.ShapeDtypeStruct((rows, n_cols), jnp.float32),
    )(idx)
    return out
